# Initial kernel scaffold; baseline (speedup 1.0000x reference)
import jax
import jax.numpy as jnp
from jax.experimental import pallas as pl


def kernel(rot_v, edge_index, rot_meas, beta_w, beta_b, lin_w, lin_b):
    raise NotImplementedError("write your pallas kernel here")


# speedup vs baseline: 2.6253x; 2.6253x over previous
import functools

import jax
import jax.numpy as jnp
from jax import lax
from jax.experimental import pallas as pl
from jax.experimental.pallas import tpu as pltpu
from jax.experimental.pallas import tpu_sc as plsc

N_NODES = 50000
E_EDGES = 1600000
NC = 2
NS = 16
NW = NC * NS
L = 16
CH = 512
SUB = 128
NCHT = E_EDGES // CH
G = CH // L


def _sc_edge_sum(table, src, dst, meas, params):
    mesh = plsc.VectorSubcoreMesh(core_axis_name="c", subcore_axis_name="s")

    @functools.partial(
        pl.kernel,
        out_type=jax.ShapeDtypeStruct((NW, L), jnp.float32),
        mesh=mesh,
        compiler_params=pltpu.CompilerParams(
            use_tc_tiling_on_sc=False, needs_layout_passes=False),
        scratch_types=[
            pltpu.VMEM((CH,), jnp.int32),
            pltpu.VMEM((CH,), jnp.int32),
            pltpu.VMEM((CH, L), jnp.float32),
            pltpu.VMEM((CH, L), jnp.float32),
            pltpu.VMEM((CH * 9,), jnp.float32),
            pltpu.VMEM((2, L), jnp.float32),
            pltpu.VMEM((L,), jnp.float32),
            pltpu.SemaphoreType.DMA,
        ],
    )
    def k(table_h, src_h, dst_h, meas_h, par_h, out_h,
          idx_s, idx_d, rows_s, rows_d, meas_v, par_v, acc_v, sem):
        w = lax.axis_index("s") * NC + lax.axis_index("c")
        pltpu.sync_copy(par_h, par_v)
        bw = par_v[0, :]
        bb = par_v[1, :]
        iota = lax.iota(jnp.int32, L)
        iota9 = iota * 9
        cols = [jnp.full((L,), j, jnp.int32) for j in range(9)]
        half = jnp.full((L,), 0.5, jnp.float32)
        thalf = jnp.full((L,), 1.5, jnp.float32)
        one = jnp.full((L,), 1.0, jnp.float32)
        magic = jnp.full((L,), 0x5F3759DF, jnp.int32)

        n_my = (NCHT - w + NW - 1) // NW

        def chunk_body(t, acc):
            base = (w + t * NW) * CH
            pltpu.sync_copy(src_h.at[pl.ds(base, CH)], idx_s)
            pltpu.sync_copy(dst_h.at[pl.ds(base, CH)], idx_d)
            base9 = base * 9
            descs = []
            for kk in range(CH // SUB):
                o = kk * SUB
                descs.append(pltpu.async_copy(
                    table_h.at[idx_s.at[pl.ds(o, SUB)]],
                    rows_s.at[pl.ds(o, SUB)], sem))
                descs.append(pltpu.async_copy(
                    table_h.at[idx_d.at[pl.ds(o, SUB)]],
                    rows_d.at[pl.ds(o, SUB)], sem))
            descs.append(pltpu.async_copy(
                meas_h.at[pl.ds(base9, CH * 9)], meas_v, sem))
            for dsc in descs:
                dsc.wait()

            def group_body(g, acc_g):
                rid = g * L + iota
                b9 = g * (L * 9) + iota9
                a = [plsc.load_gather(rows_s, [rid, cols[j]]) for j in range(9)]
                b = [plsc.load_gather(meas_v, [b9 + j]) for j in range(9)]
                d = [plsc.load_gather(rows_d, [rid, cols[j]]) for j in range(9)]
                s = None
                for r in range(3):
                    for c in range(3):
                        m = (a[3 * r] * b[c] + a[3 * r + 1] * b[3 + c]
                             + a[3 * r + 2] * b[6 + c])
                        df = m - d[3 * r + c]
                        s = df * df if s is None else s + df * df
                y = plsc.bitcast(magic - lax.shift_right_logical(
                    plsc.bitcast(s, jnp.int32), 1), jnp.float32)
                hs = half * s
                for _ in range(3):
                    y = y * (thalf - hs * y * y)
                t_cost = s * y
                x = bw * t_cost + bb
                p = one / (one + jnp.exp(-x))
                return acc_g + p

            return lax.fori_loop(0, G, group_body, acc, unroll=False)

        acc = lax.fori_loop(0, n_my, chunk_body,
                            jnp.zeros((L,), jnp.float32), unroll=False)
        acc_v[...] = acc
        pltpu.sync_copy(acc_v, out_h.at[w])

    return k(table, src, dst, meas, params)


def kernel(rot_v, edge_index, rot_meas, beta_w, beta_b, lin_w, lin_b):
    n = rot_v.shape[0]
    table = jnp.pad(rot_v.reshape(n, 9), ((0, 0), (0, 7)))
    src = edge_index[0]
    dst = edge_index[1]
    meas = rot_meas.reshape(-1)
    params = jnp.concatenate(
        [jnp.broadcast_to(beta_w.reshape(1, 1), (1, L)),
         jnp.broadcast_to(beta_b.reshape(1, 1), (1, L))], axis=0)
    partials = _sc_edge_sum(table, src, dst, meas, params)
    mean_cost = jnp.sum(partials) / n
    return mean_cost * lin_w.T + lin_b[None, :]
